# baseline (speedup 1.0000x reference)
import functools

import jax
import jax.numpy as jnp
from jax import lax
from jax.experimental import pallas as pl
from jax.experimental.pallas import tpu as pltpu
from jax.experimental.pallas import tpu_sc as plsc

ETA = 0.5
_R, _N = 128, 32768
_NC, _NS = 2, 16
_NW = _NC * _NS
_SC_ROWS = 32
_TC_ROWS = _R - _SC_ROWS
_RPW = _SC_ROWS // _NW
_L = 16
_BIG = 2**30
_CH = 16384
_CPR = _N // _CH
_NCHUNK = _RPW * _CPR
_SB = 2048
_SBPC = _CH // _SB
_SBPR = _N // _SB
_BLK = 4096
_NBLK = _N // _BLK

_mesh = plsc.VectorSubcoreMesh(core_axis_name="c", subcore_axis_name="s")

_GATHER_DNUMS = lax.GatherDimensionNumbers(
    offset_dims=(), collapsed_slice_dims=(0,), start_index_map=(0,))


def _perm(v, idx):
    return lax.gather(v, idx[:, None], _GATHER_DNUMS, (1,),
                      unique_indices=True, indices_are_sorted=False,
                      mode=lax.GatherScatterMode.PROMISE_IN_BOUNDS)


def _all_reduce(v, op, lanes):
    for sh in (8, 4, 2, 1):
        v = op(v, _perm(v, jnp.bitwise_xor(lanes, sh)))
    return v


def _tree(vals, op):
    while len(vals) > 1:
        nxt = [op(vals[i], vals[i + 1]) for i in range(0, len(vals) - 1, 2)]
        if len(vals) % 2:
            nxt.append(vals[-1])
        vals = nxt
    return vals[0]


@functools.partial(
    pl.kernel,
    out_type=jax.ShapeDtypeStruct((_NW, _L), jnp.float32),
    mesh=_mesh,
    scratch_types=[
        pltpu.VMEM((_CH,), jnp.float32),
        pltpu.VMEM((_CH,), jnp.float32),
        pltpu.VMEM((_CH,), jnp.float32),
        pltpu.VMEM((_CH,), jnp.float32),
        pltpu.VMEM((_SBPR * _L,), jnp.float32),
        pltpu.VMEM((_SBPR * _L,), jnp.float32),
        pltpu.VMEM((_L,), jnp.float32),
        pltpu.SemaphoreType.DMA,
        pltpu.SemaphoreType.DMA,
        pltpu.SemaphoreType.DMA,
        pltpu.SemaphoreType.DMA,
    ],
)
def _camp_partials(q_hbm, e_hbm, out_hbm, qbuf0, qbuf1, ebuf0, ebuf1,
                   mq, me, obuf, sq0, sq1, se0, se1):
    wid = lax.axis_index("s") * _NC + lax.axis_index("c")
    lane = lax.iota(jnp.int32, _L)
    neg_inf = jnp.full((_L,), -jnp.inf, jnp.float32)
    zeros = jnp.zeros((_L,), jnp.float32)
    ones = jnp.ones((_L,), jnp.float32)
    qbufs, ebufs = (qbuf0, qbuf1), (ebuf0, ebuf1)
    qsems, esems = (sq0, sq1), (se0, se1)

    def chunk_copies(k):
        r, c = divmod(k, _CPR)
        slot = k % 2
        row = _TC_ROWS + wid * _RPW + r
        qc = pltpu.make_async_copy(
            q_hbm.at[row, pl.ds(c * _CH, _CH)], qbufs[slot], qsems[slot])
        ec = pltpu.make_async_copy(
            e_hbm.at[row, pl.ds(c * _CH, _CH)], ebufs[slot], esems[slot])
        return qc, ec

    def merge(buf):
        sbv = [buf[pl.ds(j * _L, _L)] for j in range(_SBPR)]
        mv = _all_reduce(_tree(sbv, jnp.maximum), jnp.maximum, lane)
        perj = _tree(
            [jnp.where(v == mv, jnp.full((_L,), j, jnp.int32), _BIG)
             for j, v in enumerate(sbv)],
            jnp.minimum)
        jv = _all_reduce(perj, jnp.minimum, lane)
        excl = [jnp.where(jv == j, neg_inf, v) for j, v in enumerate(sbv)]
        secv = _all_reduce(_tree(excl, jnp.maximum), jnp.maximum, lane)
        return mv, jv, secv

    rowinfo = []
    qc, ec = chunk_copies(0)
    qc.start()
    ec.start()
    for k in range(_NCHUNK):
        r, cir = divmod(k, _CPR)
        slot = k % 2
        if k + 1 < _NCHUNK:
            nqc, nec = chunk_copies(k + 1)
            nqc.start()
            nec.start()
        qc, ec = chunk_copies(k)
        qc.wait()
        ec.wait()
        qb, eb = qbufs[slot], ebufs[slot]

        def sub_block(sbl, _, qb=qb, eb=eb, cir=cir):
            def step(t, carry, qb=qb, eb=eb, sbl=sbl):
                m1a, m1b, ema, emb = carry
                base = sbl * _SB + t * (2 * _L)
                xa = qb[pl.ds(base, _L)]
                xb = qb[pl.ds(base + _L, _L)]
                ya = eb[pl.ds(base, _L)]
                yb = eb[pl.ds(base + _L, _L)]
                return (jnp.maximum(m1a, xa), jnp.maximum(m1b, xb),
                        jnp.maximum(ema, ya), jnp.maximum(emb, yb))

            m1a, m1b, ema, emb = lax.fori_loop(
                0, _SB // (2 * _L), step,
                (neg_inf, neg_inf, neg_inf, neg_inf), unroll=8)
            sbidx = cir * _SBPC + sbl
            mq[pl.ds(sbidx * _L, _L)] = jnp.maximum(m1a, m1b)
            me[pl.ds(sbidx * _L, _L)] = jnp.maximum(ema, emb)
            return 0

        lax.fori_loop(0, _SBPC, sub_block, 0)

        if cir == _CPR - 1:
            mv, jqv, secamongv = merge(mq)
            mev, jev, _ = merge(me)
            row = _TC_ROWS + wid * _RPW + r
            rowinfo.append((row, mv, jqv, secamongv, mev, jev))

    def rescan_copies(i):
        row, _, jqv, _, _, jev = rowinfo[i]
        jq = jqv[0]
        je = jev[0]
        qc = pltpu.make_async_copy(
            q_hbm.at[row, pl.ds(jq * _SB, _SB)],
            qbuf0.at[pl.ds(i * _SB, _SB)], sq0)
        ec = pltpu.make_async_copy(
            e_hbm.at[row, pl.ds(je * _SB, _SB)],
            ebuf0.at[pl.ds(i * _SB, _SB)], se0)
        return qc, ec

    for i in range(_RPW):
        qc, ec = rescan_copies(i)
        qc.start()
        ec.start()
    for i in range(_RPW):
        qc, ec = rescan_copies(i)
        qc.wait()
        ec.wait()

    ssum = zeros
    scnt = zeros
    for i in range(_RPW):
        row, mv, jqv, secamongv, mev, jev = rowinfo[i]

        def qscan(t, carry, i=i, mv=mv):
            fid, ex, nh, idxv = carry
            x = qbuf0[pl.ds(i * _SB + t * _L, _L)]
            hit = x == mv
            fid = jnp.minimum(fid, jnp.where(hit, idxv, _BIG))
            ex = jnp.maximum(ex, jnp.where(hit, neg_inf, x))
            nh = nh + jnp.where(hit, ones, zeros)
            return fid, ex, nh, idxv + _L

        idx0 = jqv * _SB + lane
        fid, ex, nh, _ = lax.fori_loop(
            0, _SB // _L, qscan,
            (jnp.full((_L,), _BIG, jnp.int32), neg_inf, zeros, idx0),
            unroll=8)
        i1v = _all_reduce(fid, jnp.minimum, lane)
        nhv = _all_reduce(nh, jnp.add, lane)
        exv = _all_reduce(ex, jnp.maximum, lane)
        withinv = jnp.where(nhv >= 2.0, mv, exv)
        secondv = jnp.maximum(secamongv, withinv)

        def escan(t, carry, i=i, mev=mev):
            fid, idxv = carry
            x = ebuf0[pl.ds(i * _SB + t * _L, _L)]
            hit = x == mev
            fid = jnp.minimum(fid, jnp.where(hit, idxv, _BIG))
            return fid, idxv + _L

        eidx0 = jev * _SB + lane
        fide, _ = lax.fori_loop(
            0, _SB // _L, escan,
            (jnp.full((_L,), _BIG, jnp.int32), eidx0), unroll=8)
        eiv = _all_reduce(fide, jnp.minimum, lane)

        gapv = secondv - mv
        selv = (i1v == eiv) & (jnp.abs(gapv) <= ETA)
        ssum = ssum + jnp.where(selv, gapv + ETA, zeros)
        scnt = scnt + jnp.where(selv, ones, zeros)

    obuf[...] = jnp.where(lane == 0, ssum, jnp.where(lane == 1, scnt, zeros))
    pltpu.sync_copy(obuf, out_hbm.at[wid])


def _tc_body(q_ref, e_ref, vals_ref, cnts_ref, m1, m2, i1, em, ei):
    step = pl.program_id(0)
    p = _BLK // 128
    x3 = q_ref[...].reshape(_TC_ROWS, p, 128)
    y3 = e_ref[...].reshape(_TC_ROWS, p, 128)
    base = step * _BLK
    idx3 = (base
            + lax.broadcasted_iota(jnp.int32, x3.shape, 1) * 128
            + lax.broadcasted_iota(jnp.int32, x3.shape, 2))
    neg = jnp.float32(-jnp.inf)

    m1b = jnp.max(x3, axis=1)
    i1b = jnp.min(jnp.where(x3 == m1b[:, None, :], idx3, _BIG), axis=1)
    m2b = jnp.max(jnp.where(idx3 == i1b[:, None, :], neg, x3), axis=1)
    emb = jnp.max(y3, axis=1)
    eib = jnp.min(jnp.where(y3 == emb[:, None, :], idx3, _BIG), axis=1)

    @pl.when(step == 0)
    def _():
        m1[...] = m1b
        m2[...] = m2b
        i1[...] = i1b
        em[...] = emb
        ei[...] = eib

    @pl.when(step > 0)
    def _():
        pm1, pm2, pi1, pem, pei = m1[...], m2[...], i1[...], em[...], ei[...]
        gt = m1b > pm1
        m1[...] = jnp.maximum(pm1, m1b)
        i1[...] = jnp.where(gt, i1b, pi1)
        m2[...] = jnp.maximum(jnp.maximum(pm2, m2b), jnp.minimum(pm1, m1b))
        ge = emb > pem
        em[...] = jnp.maximum(pem, emb)
        ei[...] = jnp.where(ge, eib, pei)

    @pl.when(step == _NBLK - 1)
    def _():
        fm1, fm2, fi1, fem, fei = m1[...], m2[...], i1[...], em[...], ei[...]
        mrow = jnp.max(fm1, axis=1)
        irow = jnp.min(jnp.where(fm1 == mrow[:, None], fi1, _BIG), axis=1)
        m1ex = jnp.where(fi1 == irow[:, None], neg, fm1)
        second = jnp.maximum(jnp.max(fm2, axis=1), jnp.max(m1ex, axis=1))
        erow = jnp.max(fem, axis=1)
        eirow = jnp.min(jnp.where(fem == erow[:, None], fei, _BIG), axis=1)
        gap = second - mrow
        sel = (irow == eirow) & (jnp.abs(gap) <= ETA)
        vals_ref[...] = jnp.where(sel, gap + ETA, 0.0)
        cnts_ref[...] = jnp.where(sel, 1.0, 0.0)


_tc_call = pl.pallas_call(
    _tc_body,
    grid=(_NBLK,),
    in_specs=[
        pl.BlockSpec((_TC_ROWS, _BLK), lambda i: (0, i)),
        pl.BlockSpec((_TC_ROWS, _BLK), lambda i: (0, i)),
    ],
    out_specs=[
        pl.BlockSpec((_TC_ROWS,), lambda i: (0,)),
        pl.BlockSpec((_TC_ROWS,), lambda i: (0,)),
    ],
    out_shape=[
        jax.ShapeDtypeStruct((_TC_ROWS,), jnp.float32),
        jax.ShapeDtypeStruct((_TC_ROWS,), jnp.float32),
    ],
    scratch_shapes=[
        pltpu.VMEM((_TC_ROWS, 128), jnp.float32),
        pltpu.VMEM((_TC_ROWS, 128), jnp.float32),
        pltpu.VMEM((_TC_ROWS, 128), jnp.int32),
        pltpu.VMEM((_TC_ROWS, 128), jnp.float32),
        pltpu.VMEM((_TC_ROWS, 128), jnp.int32),
    ],
)


def kernel(q_table, expected_q_table):
    sc = _camp_partials(q_table, expected_q_table)
    vals, cnts = _tc_call(q_table[:_TC_ROWS], expected_q_table[:_TC_ROWS])
    s = jnp.sum(sc[:, 0]) + jnp.sum(vals)
    c = jnp.sum(sc[:, 1]) + jnp.sum(cnts)
    return jnp.where(c > 0, s / jnp.maximum(c, 1.0), 0.0)


# speedup vs baseline: 2.2416x; 2.2416x over previous
import functools

import jax
import jax.numpy as jnp
from jax import lax
from jax.experimental import pallas as pl
from jax.experimental.pallas import tpu as pltpu
from jax.experimental.pallas import tpu_sc as plsc

ETA = 0.5
_R, _N = 128, 32768
_NC, _NS = 2, 16
_NW = _NC * _NS
_SC_ROWS = 0
_TC_ROWS = _R - _SC_ROWS
_RPW = _SC_ROWS // _NW
_L = 16
_BIG = 2**30
_CH = 16384
_CPR = _N // _CH
_NCHUNK = _RPW * _CPR
_SB = 2048
_SBPC = _CH // _SB
_SBPR = _N // _SB
_BLK = 4096
_NBLK = _N // _BLK

_mesh = plsc.VectorSubcoreMesh(core_axis_name="c", subcore_axis_name="s")

_GATHER_DNUMS = lax.GatherDimensionNumbers(
    offset_dims=(), collapsed_slice_dims=(0,), start_index_map=(0,))


def _perm(v, idx):
    return lax.gather(v, idx[:, None], _GATHER_DNUMS, (1,),
                      unique_indices=True, indices_are_sorted=False,
                      mode=lax.GatherScatterMode.PROMISE_IN_BOUNDS)


def _all_reduce(v, op, lanes):
    for sh in (8, 4, 2, 1):
        v = op(v, _perm(v, jnp.bitwise_xor(lanes, sh)))
    return v


def _tree(vals, op):
    while len(vals) > 1:
        nxt = [op(vals[i], vals[i + 1]) for i in range(0, len(vals) - 1, 2)]
        if len(vals) % 2:
            nxt.append(vals[-1])
        vals = nxt
    return vals[0]


@functools.partial(
    pl.kernel,
    out_type=jax.ShapeDtypeStruct((_NW, _L), jnp.float32),
    mesh=_mesh,
    scratch_types=[
        pltpu.VMEM((_CH,), jnp.float32),
        pltpu.VMEM((_CH,), jnp.float32),
        pltpu.VMEM((_CH,), jnp.float32),
        pltpu.VMEM((_CH,), jnp.float32),
        pltpu.VMEM((_SBPR * _L,), jnp.float32),
        pltpu.VMEM((_SBPR * _L,), jnp.float32),
        pltpu.VMEM((_L,), jnp.float32),
        pltpu.SemaphoreType.DMA,
        pltpu.SemaphoreType.DMA,
        pltpu.SemaphoreType.DMA,
        pltpu.SemaphoreType.DMA,
    ],
)
def _camp_partials(q_hbm, e_hbm, out_hbm, qbuf0, qbuf1, ebuf0, ebuf1,
                   mq, me, obuf, sq0, sq1, se0, se1):
    wid = lax.axis_index("s") * _NC + lax.axis_index("c")
    lane = lax.iota(jnp.int32, _L)
    neg_inf = jnp.full((_L,), -jnp.inf, jnp.float32)
    zeros = jnp.zeros((_L,), jnp.float32)
    ones = jnp.ones((_L,), jnp.float32)
    qbufs, ebufs = (qbuf0, qbuf1), (ebuf0, ebuf1)
    qsems, esems = (sq0, sq1), (se0, se1)

    def chunk_copies(k):
        r, c = divmod(k, _CPR)
        slot = k % 2
        row = _TC_ROWS + wid * _RPW + r
        qc = pltpu.make_async_copy(
            q_hbm.at[row, pl.ds(c * _CH, _CH)], qbufs[slot], qsems[slot])
        ec = pltpu.make_async_copy(
            e_hbm.at[row, pl.ds(c * _CH, _CH)], ebufs[slot], esems[slot])
        return qc, ec

    def merge(buf):
        sbv = [buf[pl.ds(j * _L, _L)] for j in range(_SBPR)]
        mv = _all_reduce(_tree(sbv, jnp.maximum), jnp.maximum, lane)
        perj = _tree(
            [jnp.where(v == mv, jnp.full((_L,), j, jnp.int32), _BIG)
             for j, v in enumerate(sbv)],
            jnp.minimum)
        jv = _all_reduce(perj, jnp.minimum, lane)
        excl = [jnp.where(jv == j, neg_inf, v) for j, v in enumerate(sbv)]
        secv = _all_reduce(_tree(excl, jnp.maximum), jnp.maximum, lane)
        return mv, jv, secv

    rowinfo = []
    qc, ec = chunk_copies(0)
    qc.start()
    ec.start()
    for k in range(_NCHUNK):
        r, cir = divmod(k, _CPR)
        slot = k % 2
        if k + 1 < _NCHUNK:
            nqc, nec = chunk_copies(k + 1)
            nqc.start()
            nec.start()
        qc, ec = chunk_copies(k)
        qc.wait()
        ec.wait()
        qb, eb = qbufs[slot], ebufs[slot]

        def sub_block(sbl, _, qb=qb, eb=eb, cir=cir):
            def step(t, carry, qb=qb, eb=eb, sbl=sbl):
                m1a, m1b, ema, emb = carry
                base = sbl * _SB + t * (2 * _L)
                xa = qb[pl.ds(base, _L)]
                xb = qb[pl.ds(base + _L, _L)]
                ya = eb[pl.ds(base, _L)]
                yb = eb[pl.ds(base + _L, _L)]
                return (jnp.maximum(m1a, xa), jnp.maximum(m1b, xb),
                        jnp.maximum(ema, ya), jnp.maximum(emb, yb))

            m1a, m1b, ema, emb = lax.fori_loop(
                0, _SB // (2 * _L), step,
                (neg_inf, neg_inf, neg_inf, neg_inf), unroll=8)
            sbidx = cir * _SBPC + sbl
            mq[pl.ds(sbidx * _L, _L)] = jnp.maximum(m1a, m1b)
            me[pl.ds(sbidx * _L, _L)] = jnp.maximum(ema, emb)
            return 0

        lax.fori_loop(0, _SBPC, sub_block, 0)

        if cir == _CPR - 1:
            mv, jqv, secamongv = merge(mq)
            mev, jev, _ = merge(me)
            row = _TC_ROWS + wid * _RPW + r
            rowinfo.append((row, mv, jqv, secamongv, mev, jev))

    def rescan_copies(i):
        row, _, jqv, _, _, jev = rowinfo[i]
        jq = jqv[0]
        je = jev[0]
        qc = pltpu.make_async_copy(
            q_hbm.at[row, pl.ds(jq * _SB, _SB)],
            qbuf0.at[pl.ds(i * _SB, _SB)], sq0)
        ec = pltpu.make_async_copy(
            e_hbm.at[row, pl.ds(je * _SB, _SB)],
            ebuf0.at[pl.ds(i * _SB, _SB)], se0)
        return qc, ec

    for i in range(_RPW):
        qc, ec = rescan_copies(i)
        qc.start()
        ec.start()
    for i in range(_RPW):
        qc, ec = rescan_copies(i)
        qc.wait()
        ec.wait()

    ssum = zeros
    scnt = zeros
    for i in range(_RPW):
        row, mv, jqv, secamongv, mev, jev = rowinfo[i]

        def qscan(t, carry, i=i, mv=mv):
            fid, ex, nh, idxv = carry
            x = qbuf0[pl.ds(i * _SB + t * _L, _L)]
            hit = x == mv
            fid = jnp.minimum(fid, jnp.where(hit, idxv, _BIG))
            ex = jnp.maximum(ex, jnp.where(hit, neg_inf, x))
            nh = nh + jnp.where(hit, ones, zeros)
            return fid, ex, nh, idxv + _L

        idx0 = jqv * _SB + lane
        fid, ex, nh, _ = lax.fori_loop(
            0, _SB // _L, qscan,
            (jnp.full((_L,), _BIG, jnp.int32), neg_inf, zeros, idx0),
            unroll=8)
        i1v = _all_reduce(fid, jnp.minimum, lane)
        nhv = _all_reduce(nh, jnp.add, lane)
        exv = _all_reduce(ex, jnp.maximum, lane)
        withinv = jnp.where(nhv >= 2.0, mv, exv)
        secondv = jnp.maximum(secamongv, withinv)

        def escan(t, carry, i=i, mev=mev):
            fid, idxv = carry
            x = ebuf0[pl.ds(i * _SB + t * _L, _L)]
            hit = x == mev
            fid = jnp.minimum(fid, jnp.where(hit, idxv, _BIG))
            return fid, idxv + _L

        eidx0 = jev * _SB + lane
        fide, _ = lax.fori_loop(
            0, _SB // _L, escan,
            (jnp.full((_L,), _BIG, jnp.int32), eidx0), unroll=8)
        eiv = _all_reduce(fide, jnp.minimum, lane)

        gapv = secondv - mv
        selv = (i1v == eiv) & (jnp.abs(gapv) <= ETA)
        ssum = ssum + jnp.where(selv, gapv + ETA, zeros)
        scnt = scnt + jnp.where(selv, ones, zeros)

    obuf[...] = jnp.where(lane == 0, ssum, jnp.where(lane == 1, scnt, zeros))
    pltpu.sync_copy(obuf, out_hbm.at[wid])


def _tc_body(q_ref, e_ref, vals_ref, cnts_ref, m1, m2, i1, em, ei):
    step = pl.program_id(0)
    p = _BLK // 128
    x3 = q_ref[...].reshape(_TC_ROWS, p, 128)
    y3 = e_ref[...].reshape(_TC_ROWS, p, 128)
    base = step * _BLK
    idx3 = (base
            + lax.broadcasted_iota(jnp.int32, x3.shape, 1) * 128
            + lax.broadcasted_iota(jnp.int32, x3.shape, 2))
    neg = jnp.float32(-jnp.inf)

    m1b = jnp.max(x3, axis=1)
    i1b = jnp.min(jnp.where(x3 == m1b[:, None, :], idx3, _BIG), axis=1)
    m2b = jnp.max(jnp.where(idx3 == i1b[:, None, :], neg, x3), axis=1)
    emb = jnp.max(y3, axis=1)
    eib = jnp.min(jnp.where(y3 == emb[:, None, :], idx3, _BIG), axis=1)

    @pl.when(step == 0)
    def _():
        m1[...] = m1b
        m2[...] = m2b
        i1[...] = i1b
        em[...] = emb
        ei[...] = eib

    @pl.when(step > 0)
    def _():
        pm1, pm2, pi1, pem, pei = m1[...], m2[...], i1[...], em[...], ei[...]
        gt = m1b > pm1
        m1[...] = jnp.maximum(pm1, m1b)
        i1[...] = jnp.where(gt, i1b, pi1)
        m2[...] = jnp.maximum(jnp.maximum(pm2, m2b), jnp.minimum(pm1, m1b))
        ge = emb > pem
        em[...] = jnp.maximum(pem, emb)
        ei[...] = jnp.where(ge, eib, pei)

    @pl.when(step == _NBLK - 1)
    def _():
        fm1, fm2, fi1, fem, fei = m1[...], m2[...], i1[...], em[...], ei[...]
        mrow = jnp.max(fm1, axis=1)
        irow = jnp.min(jnp.where(fm1 == mrow[:, None], fi1, _BIG), axis=1)
        m1ex = jnp.where(fi1 == irow[:, None], neg, fm1)
        second = jnp.maximum(jnp.max(fm2, axis=1), jnp.max(m1ex, axis=1))
        erow = jnp.max(fem, axis=1)
        eirow = jnp.min(jnp.where(fem == erow[:, None], fei, _BIG), axis=1)
        gap = second - mrow
        sel = (irow == eirow) & (jnp.abs(gap) <= ETA)
        vals_ref[...] = jnp.where(sel, gap + ETA, 0.0)
        cnts_ref[...] = jnp.where(sel, 1.0, 0.0)


_tc_call = pl.pallas_call(
    _tc_body,
    grid=(_NBLK,),
    in_specs=[
        pl.BlockSpec((_TC_ROWS, _BLK), lambda i: (0, i)),
        pl.BlockSpec((_TC_ROWS, _BLK), lambda i: (0, i)),
    ],
    out_specs=[
        pl.BlockSpec((_TC_ROWS,), lambda i: (0,)),
        pl.BlockSpec((_TC_ROWS,), lambda i: (0,)),
    ],
    out_shape=[
        jax.ShapeDtypeStruct((_TC_ROWS,), jnp.float32),
        jax.ShapeDtypeStruct((_TC_ROWS,), jnp.float32),
    ],
    scratch_shapes=[
        pltpu.VMEM((_TC_ROWS, 128), jnp.float32),
        pltpu.VMEM((_TC_ROWS, 128), jnp.float32),
        pltpu.VMEM((_TC_ROWS, 128), jnp.int32),
        pltpu.VMEM((_TC_ROWS, 128), jnp.float32),
        pltpu.VMEM((_TC_ROWS, 128), jnp.int32),
    ],
)


def kernel(q_table, expected_q_table):
    vals, cnts = _tc_call(q_table[:_TC_ROWS], expected_q_table[:_TC_ROWS])
    s = jnp.sum(vals)
    c = jnp.sum(cnts)
    return jnp.where(c > 0, s / jnp.maximum(c, 1.0), 0.0)
